# Initial kernel scaffold; baseline (speedup 1.0000x reference)
import jax
import jax.numpy as jnp
from jax.experimental import pallas as pl


def kernel(x, edge_index, edge_type, W1, root1, b1, W2, root2, b2):
    raise NotImplementedError("write your pallas kernel here")


# speedup vs baseline: 3.3470x; 3.3470x over previous
import jax
import jax.numpy as jnp
from jax import lax
from jax.experimental import pallas as pl
from jax.experimental.pallas import tpu as pltpu
from jax.experimental.pallas import tpu_sc as plsc

NN = 100000
NE = 3200000
R = 8
INCH = 16
HID = 32
OUTCH = 2

NC = 2
NS = 16
L = 16

BLK = 128
NBUF = 8
NBLKS_P = 25088
NE_P = NBLKS_P * BLK
PER_SUB = NBLKS_P // NS
EROW = 3 * BLK

HALF = NN // NC
ACC_N_ROWS = 65536
DUMP_N = HALF

CHUNK1 = 12500
NCHUNK1 = NN // CHUNK1
ACC1_ROWS = 100352
DUMP1 = CHUNK1 * R

_MESH = plsc.VectorSubcoreMesh(core_axis_name="c", subcore_axis_name="s")
_SC_PARAMS = pltpu.CompilerParams(use_tc_tiling_on_sc=False)


def _edge_scan(sid, base, hi, *, e3_h, table_h, acc, bufs, sems,
               smode, gmode, dump):
    ebuf, gidx, sidx, rows = bufs
    start = sid * PER_SUB

    def group_body(g, carry):
        b0 = start + g * NBUF
        lds, gds, sds = [], [], []
        for u in range(NBUF):
            d = pltpu.make_async_copy(e3_h.at[b0 + u], ebuf[u],
                                      sems.at[u, 0])
            d.start()
            lds.append(d)
        for u in range(NBUF):
            lds[u].wait()
            for j in range(BLK // L):
                sl = pl.ds(j * L, L)
                dv = ebuf[u][pl.ds(BLK + j * L, L)]
                ok = (dv >= base) & (dv < base + hi)
                if smode == "node_rel":
                    ev = ebuf[u][pl.ds(2 * BLK + j * L, L)]
                    s = jnp.where(ok, (dv - base) * R + ev, dump)
                else:
                    s = jnp.where(ok, dv - base, dump)
                sidx[u][sl] = s
                if gmode == "et":
                    gidx[u][sl] = ebuf[u][pl.ds(2 * BLK + j * L, L)]
                elif gmode == "src":
                    gidx[u][sl] = ebuf[u][pl.ds(j * L, L)]
                else:
                    gidx[u][sl] = (ebuf[u][pl.ds(j * L, L)] * R
                                   + ebuf[u][pl.ds(2 * BLK + j * L, L)])
            d = pltpu.make_async_copy(table_h.at[gidx[u]], rows[u],
                                      sems.at[u, 1])
            d.start()
            gds.append(d)
        for u in range(NBUF):
            gds[u].wait()
            d = pltpu.make_async_copy(rows[u], acc.at[sidx[u]],
                                      sems.at[u, 2])
            d.start(add=True)
            sds.append(d)
        for u in range(NBUF):
            sds[u].wait()
        return carry

    lax.fori_loop(0, PER_SUB // NBUF, group_body, 0)


def _zero_acc(zb, acc, sid, per_tile):
    def zchunk(k, carry):
        pltpu.sync_copy(zb, acc.at[pl.ds(sid * per_tile + k * 128, 128)])
        return carry

    lax.fori_loop(0, per_tile // 128, zchunk, 0)


_ZROWS_N = 4096
_ZROWS_1 = 6272


def _flush(acc, out_h, out_base, total, sid):
    s_main = (total // NS) & ~7
    last = total - s_main * (NS - 1)

    @pl.when(sid < NS - 1)
    def _():
        pltpu.sync_copy(
            acc.at[pl.ds(sid * s_main, s_main)],
            out_h.at[pl.ds(out_base + sid * s_main, s_main)],
        )

    @pl.when(sid == NS - 1)
    def _():
        pltpu.sync_copy(
            acc.at[pl.ds((NS - 1) * s_main, last)],
            out_h.at[pl.ds(out_base + (NS - 1) * s_main, last)],
        )


def _scratch_common(acc_rows):
    return [
        pltpu.VMEM_SHARED((acc_rows, L), jnp.float32),
        pltpu.VMEM((128, L), jnp.float32),
        [pltpu.VMEM((EROW,), jnp.int32) for _ in range(NBUF)],
        [pltpu.VMEM((BLK,), jnp.int32) for _ in range(NBUF)],
        [pltpu.VMEM((BLK,), jnp.int32) for _ in range(NBUF)],
        [pltpu.VMEM((BLK, L), jnp.float32) for _ in range(NBUF)],
        pltpu.SemaphoreType.DMA((NBUF, 3)),
    ]


def _counts_body(e3_h, onehot_h, zeros_h, cnt_h,
                 acc, zb, ebuf, gidx, sidx, rows, sems):
    cid = lax.axis_index("c")
    sid = lax.axis_index("s")
    base = cid * HALF
    pltpu.sync_copy(zeros_h, zb)
    _zero_acc(zb, acc, sid, _ZROWS_N)
    plsc.subcore_barrier()
    _edge_scan(sid, base, HALF, e3_h=e3_h, table_h=onehot_h, acc=acc,
               bufs=(ebuf, gidx, sidx, rows), sems=sems,
               smode="node", gmode="et", dump=DUMP_N)
    plsc.subcore_barrier()
    _flush(acc, cnt_h, base, HALF, sid)


def _l1_body(e3_h, x_h, zeros_h, s1_h,
             acc, zb, ebuf, gidx, sidx, rows, sems):
    cid = lax.axis_index("c")
    sid = lax.axis_index("s")
    pltpu.sync_copy(zeros_h, zb)

    def pass_body(p, carry):
        chunk = cid * (NCHUNK1 // NC) + p
        base = chunk * CHUNK1
        _zero_acc(zb, acc, sid, _ZROWS_1)
        plsc.subcore_barrier()
        _edge_scan(sid, base, CHUNK1, e3_h=e3_h, table_h=x_h, acc=acc,
                   bufs=(ebuf, gidx, sidx, rows), sems=sems,
                   smode="node_rel", gmode="src", dump=DUMP1)
        plsc.subcore_barrier()
        _flush(acc, s1_h, chunk * CHUNK1 * R, CHUNK1 * R, sid)
        plsc.subcore_barrier()
        return carry

    lax.fori_loop(0, NCHUNK1 // NC, pass_body, 0)


def _l2_body(e3_h, zm_h, zeros_h, s2_h,
             acc, zb, ebuf, gidx, sidx, rows, sems):
    cid = lax.axis_index("c")
    sid = lax.axis_index("s")
    base = cid * HALF
    pltpu.sync_copy(zeros_h, zb)
    _zero_acc(zb, acc, sid, _ZROWS_N)
    plsc.subcore_barrier()
    _edge_scan(sid, base, HALF, e3_h=e3_h, table_h=zm_h, acc=acc,
               bufs=(ebuf, gidx, sidx, rows), sems=sems,
               smode="node", gmode="src_rel", dump=DUMP_N)
    plsc.subcore_barrier()
    _flush(acc, s2_h, base, HALF, sid)


_counts_call = pl.kernel(
    _counts_body,
    out_type=jax.ShapeDtypeStruct((NN, L), jnp.float32),
    mesh=_MESH,
    scratch_types=_scratch_common(ACC_N_ROWS),
    compiler_params=_SC_PARAMS,
)

_l1_call = pl.kernel(
    _l1_body,
    out_type=jax.ShapeDtypeStruct((NN * R, L), jnp.float32),
    mesh=_MESH,
    scratch_types=_scratch_common(ACC1_ROWS),
    compiler_params=_SC_PARAMS,
)

_l2_call = pl.kernel(
    _l2_body,
    out_type=jax.ShapeDtypeStruct((NN, L), jnp.float32),
    mesh=_MESH,
    scratch_types=_scratch_common(ACC_N_ROWS),
    compiler_params=_SC_PARAMS,
)


BROW = 2000


def _dense1_body(x_ref, s1_ref, cnt_ref, w1_ref, r1_ref, b1_ref, w2c_ref,
                 h_ref, zm_ref):
    xb = x_ref[...]
    s1 = s1_ref[...]
    cb = jnp.maximum(cnt_ref[...][:, :R], 1.0)
    acc = jnp.dot(xb, r1_ref[...], preferred_element_type=jnp.float32)
    acc = acc + b1_ref[...]
    for r in range(R):
        pr = jnp.dot(s1[:, r * INCH:(r + 1) * INCH],
                     w1_ref[...][r * INCH:(r + 1) * INCH, :],
                     preferred_element_type=jnp.float32)
        acc = acc + pr / cb[:, r:r + 1]
    h = jnp.maximum(acc, 0.0)
    h_ref[...] = h
    z = jnp.dot(h, w2c_ref[...], preferred_element_type=jnp.float32)
    zt = jnp.concatenate([z] * R, axis=1)
    lane = lax.broadcasted_iota(jnp.int32, (BROW, R * L), 1)
    keep = (lane % L) // OUTCH == lane // L
    zm_ref[...] = jnp.where(keep, zt, 0.0)


def _dense2_body(h_ref, s2_ref, cnt_ref, r2_ref, b2_ref, out_ref):
    h = h_ref[...]
    s2 = s2_ref[...]
    cb = jnp.maximum(cnt_ref[...][:, :R], 1.0)
    acc = jnp.dot(h, r2_ref[...], preferred_element_type=jnp.float32)
    acc = acc + b2_ref[...]
    for r in range(R):
        acc = acc + s2[:, OUTCH * r:OUTCH * (r + 1)] / cb[:, r:r + 1]
    m = jnp.max(acc, axis=1, keepdims=True)
    ex = jnp.exp(acc - m)
    out_ref[...] = acc - m - jnp.log(jnp.sum(ex, axis=1, keepdims=True))


def _full_spec(shape):
    return pl.BlockSpec(shape, lambda i: (0, 0))


_dense1_call = pl.pallas_call(
    _dense1_body,
    grid=(NN // BROW,),
    in_specs=[
        pl.BlockSpec((BROW, INCH), lambda i: (i, 0)),
        pl.BlockSpec((BROW, R * INCH), lambda i: (i, 0)),
        pl.BlockSpec((BROW, L), lambda i: (i, 0)),
        _full_spec((R * INCH, HID)),
        _full_spec((INCH, HID)),
        _full_spec((1, HID)),
        _full_spec((HID, R * OUTCH)),
    ],
    out_specs=[
        pl.BlockSpec((BROW, HID), lambda i: (i, 0)),
        pl.BlockSpec((BROW, R * L), lambda i: (i, 0)),
    ],
    out_shape=[
        jax.ShapeDtypeStruct((NN, HID), jnp.float32),
        jax.ShapeDtypeStruct((NN, R * L), jnp.float32),
    ],
)

_dense2_call = pl.pallas_call(
    _dense2_body,
    grid=(NN // BROW,),
    in_specs=[
        pl.BlockSpec((BROW, HID), lambda i: (i, 0)),
        pl.BlockSpec((BROW, L), lambda i: (i, 0)),
        pl.BlockSpec((BROW, L), lambda i: (i, 0)),
        _full_spec((HID, OUTCH)),
        _full_spec((1, OUTCH)),
    ],
    out_specs=pl.BlockSpec((BROW, OUTCH), lambda i: (i, 0)),
    out_shape=jax.ShapeDtypeStruct((NN, OUTCH), jnp.float32),
)


def kernel(x, edge_index, edge_type, W1, root1, b1, W2, root2, b2):
    pad = NE_P - NE
    src_p = jnp.concatenate([edge_index[0], jnp.zeros((pad,), jnp.int32)])
    dst_p = jnp.concatenate([edge_index[1], jnp.full((pad,), NN, jnp.int32)])
    et_p = jnp.concatenate([edge_type, jnp.zeros((pad,), jnp.int32)])
    e3 = jnp.stack([src_p.reshape(NBLKS_P, BLK), dst_p.reshape(NBLKS_P, BLK),
                    et_p.reshape(NBLKS_P, BLK)], axis=1).reshape(NBLKS_P, EROW)

    onehot = jnp.eye(R, L, dtype=jnp.float32)
    zeros128 = jnp.zeros((128, L), jnp.float32)

    cnt = _counts_call(e3, onehot, zeros128)
    s1 = _l1_call(e3, x, zeros128)

    w1cat = W1.reshape(R * INCH, HID)
    w2cat = jnp.transpose(W2, (1, 0, 2)).reshape(HID, R * OUTCH)
    h, zm = _dense1_call(x, s1.reshape(NN, R * INCH), cnt, w1cat, root1,
                         b1.reshape(1, HID), w2cat)
    s2 = _l2_call(e3, zm.reshape(NN * R, L), zeros128)
    out = _dense2_call(h, s2, cnt, root2, b2.reshape(1, OUTCH))
    return out


# speedup vs baseline: 14.1281x; 4.2212x over previous
import jax
import jax.numpy as jnp
from jax import lax
from jax.experimental import pallas as pl
from jax.experimental.pallas import tpu as pltpu
from jax.experimental.pallas import tpu_sc as plsc

NN = 100000
NE = 3200000
R = 8
INCH = 16
HID = 32
OUTCH = 2

NC = 2
NS = 16
L = 16

BLK = 128
NBUF = 8
NBLKS_P = 25088
NE_P = NBLKS_P * BLK
PER_SUB = NBLKS_P // NS
EROW = 3 * BLK

BF = jnp.bfloat16

HALF = NN // NC
ACC_N_ROWS = 65536
DUMP_N = HALF

CHUNK1 = 12500
NCHUNK1 = NN // CHUNK1
ACC1_ROWS = 100352
DUMP1 = CHUNK1 * R

_MESH = plsc.VectorSubcoreMesh(core_axis_name="c", subcore_axis_name="s")
_SC_PARAMS = pltpu.CompilerParams(use_tc_tiling_on_sc=False)


def _edge_scan(sid, base, hi, *, e3_h, table, acc, bufs, sems,
               smode, gmode, dump):
    ebuf, gidx, sidx, rows = bufs
    start = sid * PER_SUB

    def group_body(g, carry):
        b0 = start + g * NBUF
        lds, gds, sds = [], [], []
        for u in range(NBUF):
            d = pltpu.make_async_copy(e3_h.at[b0 + u], ebuf[u],
                                      sems.at[u, 0])
            d.start()
            lds.append(d)
        for u in range(NBUF):
            lds[u].wait()
            for j in range(BLK // L):
                sl = pl.ds(j * L, L)
                dv = ebuf[u][pl.ds(BLK + j * L, L)]
                ok = (dv >= base) & (dv < base + hi)
                if smode == "node_rel":
                    ev = ebuf[u][pl.ds(2 * BLK + j * L, L)]
                    s = jnp.where(ok, (dv - base) * R + ev, dump)
                else:
                    s = jnp.where(ok, dv - base, dump)
                sidx[u][sl] = s
                if gmode == "et":
                    gidx[u][sl] = ebuf[u][pl.ds(2 * BLK + j * L, L)]
                elif gmode == "src":
                    gidx[u][sl] = ebuf[u][pl.ds(j * L, L)]
                else:
                    gidx[u][sl] = (ebuf[u][pl.ds(j * L, L)] * R
                                   + ebuf[u][pl.ds(2 * BLK + j * L, L)])
            d = pltpu.make_async_copy(table.at[gidx[u]], rows[u],
                                      sems.at[u, 1])
            d.start()
            gds.append(d)
        for u in range(NBUF):
            gds[u].wait()
            d = pltpu.make_async_copy(rows[u], acc.at[sidx[u]],
                                      sems.at[u, 2])
            d.start(add=True)
            sds.append(d)
        for u in range(NBUF):
            sds[u].wait()
        return carry

    lax.fori_loop(0, PER_SUB // NBUF, group_body, 0)


def _zero_acc(zb, acc, sid, per_tile):
    def zchunk(k, carry):
        pltpu.sync_copy(zb, acc.at[pl.ds(sid * per_tile + k * 128, 128)])
        return carry

    lax.fori_loop(0, per_tile // 128, zchunk, 0)


_ZROWS_N = 4096
_ZROWS_1 = 6272


def _striped(src_fn, dst_fn, total, sid):
    s_main = (total // NS) & ~7
    last = total - s_main * (NS - 1)

    @pl.when(sid < NS - 1)
    def _():
        pltpu.sync_copy(src_fn(sid * s_main, s_main),
                        dst_fn(sid * s_main, s_main))

    @pl.when(sid == NS - 1)
    def _():
        pltpu.sync_copy(src_fn((NS - 1) * s_main, last),
                        dst_fn((NS - 1) * s_main, last))


def _flush(acc, out_h, out_base, total, sid):
    _striped(lambda o, n: acc.at[pl.ds(o, n)],
             lambda o, n: out_h.at[pl.ds(out_base + o, n)], total, sid)


def _stage(in_h, spm, total, sid):
    _striped(lambda o, n: in_h.at[pl.ds(o, n)],
             lambda o, n: spm.at[pl.ds(o, n)], total, sid)


def _scratch_common(acc_rows, dt):
    return [
        pltpu.VMEM_SHARED((acc_rows, L), dt),
        pltpu.VMEM((128, L), dt),
        [pltpu.VMEM((EROW,), jnp.int32) for _ in range(NBUF)],
        [pltpu.VMEM((BLK,), jnp.int32) for _ in range(NBUF)],
        [pltpu.VMEM((BLK,), jnp.int32) for _ in range(NBUF)],
        [pltpu.VMEM((BLK, L), dt) for _ in range(NBUF)],
        pltpu.SemaphoreType.DMA((NBUF, 3)),
    ]


def _counts_body(e3_h, onehot_h, zeros_h, cnt_h,
                 acc, zb, ebuf, gidx, sidx, rows, sems, pm):
    cid = lax.axis_index("c")
    sid = lax.axis_index("s")
    base = cid * HALF
    pltpu.sync_copy(zeros_h, zb)

    @pl.when(sid == 0)
    def _():
        pltpu.sync_copy(onehot_h, pm)

    _zero_acc(zb, acc, sid, _ZROWS_N)
    plsc.subcore_barrier()
    _edge_scan(sid, base, HALF, e3_h=e3_h, table=pm, acc=acc,
               bufs=(ebuf, gidx, sidx, rows), sems=sems,
               smode="node", gmode="et", dump=DUMP_N)
    plsc.subcore_barrier()
    _flush(acc, cnt_h, base, HALF, sid)


def _l1_body(e3_h, x_h, zeros_h, s1_h,
             acc, zb, ebuf, gidx, sidx, rows, sems, xs):
    cid = lax.axis_index("c")
    sid = lax.axis_index("s")
    pltpu.sync_copy(zeros_h, zb)
    _stage(x_h, xs, NN, sid)

    def pass_body(p, carry):
        chunk = cid * (NCHUNK1 // NC) + p
        base = chunk * CHUNK1
        _zero_acc(zb, acc, sid, _ZROWS_1)
        plsc.subcore_barrier()
        _edge_scan(sid, base, CHUNK1, e3_h=e3_h, table=xs, acc=acc,
                   bufs=(ebuf, gidx, sidx, rows), sems=sems,
                   smode="node_rel", gmode="src", dump=DUMP1)
        plsc.subcore_barrier()
        _flush(acc, s1_h, chunk * CHUNK1 * R, CHUNK1 * R, sid)
        plsc.subcore_barrier()
        return carry

    lax.fori_loop(0, NCHUNK1 // NC, pass_body, 0)


def _l2_body(e3_h, zm_h, zeros_h, s2_h,
             acc, zb, ebuf, gidx, sidx, rows, sems):
    cid = lax.axis_index("c")
    sid = lax.axis_index("s")
    base = cid * HALF
    pltpu.sync_copy(zeros_h, zb)
    _zero_acc(zb, acc, sid, _ZROWS_N)
    plsc.subcore_barrier()
    _edge_scan(sid, base, HALF, e3_h=e3_h, table=zm_h, acc=acc,
               bufs=(ebuf, gidx, sidx, rows), sems=sems,
               smode="node", gmode="src_rel", dump=DUMP_N)
    plsc.subcore_barrier()
    _flush(acc, s2_h, base, HALF, sid)


_counts_call = pl.kernel(
    _counts_body,
    out_type=jax.ShapeDtypeStruct((NN, L), BF),
    mesh=_MESH,
    scratch_types=_scratch_common(ACC_N_ROWS, BF) + [
        pltpu.VMEM_SHARED((R, L), BF)],
    compiler_params=_SC_PARAMS,
)

_l1_call = pl.kernel(
    _l1_body,
    out_type=jax.ShapeDtypeStruct((NN * R, L), BF),
    mesh=_MESH,
    scratch_types=_scratch_common(ACC1_ROWS, BF) + [
        pltpu.VMEM_SHARED((NN, L), BF)],
    compiler_params=_SC_PARAMS,
)

_l2_call = pl.kernel(
    _l2_body,
    out_type=jax.ShapeDtypeStruct((NN, L), jnp.float32),
    mesh=_MESH,
    scratch_types=_scratch_common(ACC_N_ROWS, jnp.float32),
    compiler_params=_SC_PARAMS,
)


BROW = 2000


def _dense1_body(x_ref, s1_ref, cnt_ref, w1_ref, r1_ref, b1_ref, w2c_ref,
                 h_ref, zm_ref):
    xb = x_ref[...]
    s1 = s1_ref[...].astype(jnp.float32)
    cb = jnp.maximum(cnt_ref[...][:, :R].astype(jnp.float32), 1.0)
    acc = jnp.dot(xb, r1_ref[...], preferred_element_type=jnp.float32)
    acc = acc + b1_ref[...]
    for r in range(R):
        pr = jnp.dot(s1[:, r * INCH:(r + 1) * INCH],
                     w1_ref[...][r * INCH:(r + 1) * INCH, :],
                     preferred_element_type=jnp.float32)
        acc = acc + pr / cb[:, r:r + 1]
    h = jnp.maximum(acc, 0.0)
    h_ref[...] = h
    z = jnp.dot(h, w2c_ref[...], preferred_element_type=jnp.float32)
    zt = jnp.concatenate([z] * R, axis=1)
    lane = lax.broadcasted_iota(jnp.int32, (BROW, R * L), 1)
    keep = (lane % L) // OUTCH == lane // L
    zm_ref[...] = jnp.where(keep, zt, 0.0)


def _dense2_body(h_ref, s2_ref, cnt_ref, r2_ref, b2_ref, out_ref):
    h = h_ref[...]
    s2 = s2_ref[...].astype(jnp.float32)
    cb = jnp.maximum(cnt_ref[...][:, :R].astype(jnp.float32), 1.0)
    acc = jnp.dot(h, r2_ref[...], preferred_element_type=jnp.float32)
    acc = acc + b2_ref[...]
    for r in range(R):
        acc = acc + s2[:, OUTCH * r:OUTCH * (r + 1)] / cb[:, r:r + 1]
    m = jnp.max(acc, axis=1, keepdims=True)
    ex = jnp.exp(acc - m)
    out_ref[...] = acc - m - jnp.log(jnp.sum(ex, axis=1, keepdims=True))


def _full_spec(shape):
    return pl.BlockSpec(shape, lambda i: (0, 0))


_dense1_call = pl.pallas_call(
    _dense1_body,
    grid=(NN // BROW,),
    in_specs=[
        pl.BlockSpec((BROW, INCH), lambda i: (i, 0)),
        pl.BlockSpec((BROW, R * INCH), lambda i: (i, 0)),
        pl.BlockSpec((BROW, L), lambda i: (i, 0)),
        _full_spec((R * INCH, HID)),
        _full_spec((INCH, HID)),
        _full_spec((1, HID)),
        _full_spec((HID, R * OUTCH)),
    ],
    out_specs=[
        pl.BlockSpec((BROW, HID), lambda i: (i, 0)),
        pl.BlockSpec((BROW, R * L), lambda i: (i, 0)),
    ],
    out_shape=[
        jax.ShapeDtypeStruct((NN, HID), jnp.float32),
        jax.ShapeDtypeStruct((NN, R * L), jnp.float32),
    ],
)

_dense2_call = pl.pallas_call(
    _dense2_body,
    grid=(NN // BROW,),
    in_specs=[
        pl.BlockSpec((BROW, HID), lambda i: (i, 0)),
        pl.BlockSpec((BROW, L), lambda i: (i, 0)),
        pl.BlockSpec((BROW, L), lambda i: (i, 0)),
        _full_spec((HID, OUTCH)),
        _full_spec((1, OUTCH)),
    ],
    out_specs=pl.BlockSpec((BROW, OUTCH), lambda i: (i, 0)),
    out_shape=jax.ShapeDtypeStruct((NN, OUTCH), jnp.float32),
)


def kernel(x, edge_index, edge_type, W1, root1, b1, W2, root2, b2):
    pad = NE_P - NE
    src_p = jnp.concatenate([edge_index[0], jnp.zeros((pad,), jnp.int32)])
    dst_p = jnp.concatenate([edge_index[1], jnp.full((pad,), NN, jnp.int32)])
    et_p = jnp.concatenate([edge_type, jnp.zeros((pad,), jnp.int32)])
    e3 = jnp.stack([src_p.reshape(NBLKS_P, BLK), dst_p.reshape(NBLKS_P, BLK),
                    et_p.reshape(NBLKS_P, BLK)], axis=1).reshape(NBLKS_P, EROW)

    onehot = jnp.eye(R, L, dtype=BF)
    zeros128b = jnp.zeros((128, L), BF)
    zeros128f = jnp.zeros((128, L), jnp.float32)

    cnt = _counts_call(e3, onehot, zeros128b)
    s1 = _l1_call(e3, x.astype(BF), zeros128b)

    w1cat = W1.reshape(R * INCH, HID)
    w2cat = jnp.transpose(W2, (1, 0, 2)).reshape(HID, R * OUTCH)
    h, zm = _dense1_call(x, s1.reshape(NN, R * INCH), cnt, w1cat, root1,
                         b1.reshape(1, HID), w2cat)
    s2 = _l2_call(e3, zm.reshape(NN * R, L), zeros128f)
    out = _dense2_call(h, s2, cnt, root2, b2.reshape(1, OUTCH))
    return out


# speedup vs baseline: 16.2599x; 1.1509x over previous
import jax
import jax.numpy as jnp
from jax import lax
from jax.experimental import pallas as pl
from jax.experimental.pallas import tpu as pltpu
from jax.experimental.pallas import tpu_sc as plsc

NN = 100000
NE = 3200000
R = 8
INCH = 16
HID = 32
OUTCH = 2

NC = 2
NS = 16
L = 16

BLK = 128
NBUF = 6
NBLKS_P = 25152
NE_P = NBLKS_P * BLK
PER_SUB = NBLKS_P // NS
EROW = 3 * BLK

BF = jnp.bfloat16

HALF = NN // NC
ACC_N_ROWS = 65536
DUMP_N = HALF

CHUNK1 = 16672
NCHUNK1 = 6
NN1 = CHUNK1 * NCHUNK1
ACC1_ROWS = 135168
DUMP1 = CHUNK1 * R

_MESH = plsc.VectorSubcoreMesh(core_axis_name="c", subcore_axis_name="s")
_SC_PARAMS = pltpu.CompilerParams(use_tc_tiling_on_sc=False)


def _edge_scan(sid, base, hi, *, e3_h, table, acc, bufs, sems,
               smode, gmode, dump):
    ebuf, gidx, sidx, rows = bufs
    start = sid * PER_SUB

    def group_body(g, carry):
        b0 = start + g * NBUF
        lds, gds, sds = [], [], []
        for u in range(NBUF):
            d = pltpu.make_async_copy(e3_h.at[b0 + u], ebuf[u],
                                      sems.at[u, 0])
            d.start()
            lds.append(d)
        for u in range(NBUF):
            lds[u].wait()
            for j in range(BLK // L):
                sl = pl.ds(j * L, L)
                dv = ebuf[u][pl.ds(BLK + j * L, L)]
                ok = (dv >= base) & (dv < base + hi)
                if smode == "node_rel":
                    ev = ebuf[u][pl.ds(2 * BLK + j * L, L)]
                    s = jnp.where(ok, (dv - base) * R + ev, dump)
                else:
                    s = jnp.where(ok, dv - base, dump)
                sidx[u][sl] = s
                if gmode == "et":
                    gidx[u][sl] = ebuf[u][pl.ds(2 * BLK + j * L, L)]
                elif gmode == "src":
                    gidx[u][sl] = ebuf[u][pl.ds(j * L, L)]
                else:
                    gidx[u][sl] = (ebuf[u][pl.ds(j * L, L)] * R
                                   + ebuf[u][pl.ds(2 * BLK + j * L, L)])
            d = pltpu.make_async_copy(table.at[gidx[u]], rows[u],
                                      sems.at[u, 1])
            d.start()
            gds.append(d)
        for u in range(NBUF):
            gds[u].wait()
            d = pltpu.make_async_copy(rows[u], acc.at[sidx[u]],
                                      sems.at[u, 2])
            d.start(add=True)
            sds.append(d)
        for u in range(NBUF):
            sds[u].wait()
        return carry

    lax.fori_loop(0, PER_SUB // NBUF, group_body, 0)


def _zero_acc(zb, acc, sid, per_tile):
    def zchunk(k, carry):
        pltpu.sync_copy(zb, acc.at[pl.ds(sid * per_tile + k * 128, 128)])
        return carry

    lax.fori_loop(0, per_tile // 128, zchunk, 0)


_ZROWS_N = 4096
_ZROWS_1 = 8448


def _striped(src_fn, dst_fn, total, sid):
    s_main = (total // NS) & ~7
    last = total - s_main * (NS - 1)

    @pl.when(sid < NS - 1)
    def _():
        pltpu.sync_copy(src_fn(sid * s_main, s_main),
                        dst_fn(sid * s_main, s_main))

    @pl.when(sid == NS - 1)
    def _():
        pltpu.sync_copy(src_fn((NS - 1) * s_main, last),
                        dst_fn((NS - 1) * s_main, last))


def _flush(acc, out_h, out_base, total, sid):
    _striped(lambda o, n: acc.at[pl.ds(o, n)],
             lambda o, n: out_h.at[pl.ds(out_base + o, n)], total, sid)


def _stage(in_h, spm, total, sid):
    _striped(lambda o, n: in_h.at[pl.ds(o, n)],
             lambda o, n: spm.at[pl.ds(o, n)], total, sid)


def _scratch_common(acc_rows, dt):
    return [
        pltpu.VMEM_SHARED((acc_rows, L), dt),
        pltpu.VMEM((128, L), dt),
        [pltpu.VMEM((EROW,), jnp.int32) for _ in range(NBUF)],
        [pltpu.VMEM((BLK,), jnp.int32) for _ in range(NBUF)],
        [pltpu.VMEM((BLK,), jnp.int32) for _ in range(NBUF)],
        [pltpu.VMEM((BLK, L), dt) for _ in range(NBUF)],
        pltpu.SemaphoreType.DMA((NBUF, 3)),
    ]


def _counts_body(e3_h, onehot_h, zeros_h, cnt_h,
                 acc, zb, ebuf, gidx, sidx, rows, sems, pm):
    cid = lax.axis_index("c")
    sid = lax.axis_index("s")
    base = cid * HALF
    pltpu.sync_copy(zeros_h, zb)

    @pl.when(sid == 0)
    def _():
        pltpu.sync_copy(onehot_h, pm)

    _zero_acc(zb, acc, sid, _ZROWS_N)
    plsc.subcore_barrier()
    _edge_scan(sid, base, HALF, e3_h=e3_h, table=pm, acc=acc,
               bufs=(ebuf, gidx, sidx, rows), sems=sems,
               smode="node", gmode="et", dump=DUMP_N)
    plsc.subcore_barrier()
    _flush(acc, cnt_h, base, HALF, sid)


def _l1_body(e3_h, x_h, zeros_h, s1_h,
             acc, zb, ebuf, gidx, sidx, rows, sems, xs):
    cid = lax.axis_index("c")
    sid = lax.axis_index("s")
    pltpu.sync_copy(zeros_h, zb)
    _stage(x_h, xs, NN, sid)

    def pass_body(p, carry):
        chunk = cid * (NCHUNK1 // NC) + p
        base = chunk * CHUNK1
        _zero_acc(zb, acc, sid, _ZROWS_1)
        plsc.subcore_barrier()
        _edge_scan(sid, base, CHUNK1, e3_h=e3_h, table=xs, acc=acc,
                   bufs=(ebuf, gidx, sidx, rows), sems=sems,
                   smode="node_rel", gmode="src", dump=DUMP1)
        plsc.subcore_barrier()
        _flush(acc, s1_h, chunk * CHUNK1 * R, CHUNK1 * R, sid)
        plsc.subcore_barrier()
        return carry

    lax.fori_loop(0, NCHUNK1 // NC, pass_body, 0)


def _l2_body(e3_h, zm_h, zeros_h, s2_h,
             acc, zb, ebuf, gidx, sidx, rows, sems):
    cid = lax.axis_index("c")
    sid = lax.axis_index("s")
    base = cid * HALF
    pltpu.sync_copy(zeros_h, zb)
    _zero_acc(zb, acc, sid, _ZROWS_N)
    plsc.subcore_barrier()
    _edge_scan(sid, base, HALF, e3_h=e3_h, table=zm_h, acc=acc,
               bufs=(ebuf, gidx, sidx, rows), sems=sems,
               smode="node", gmode="src_rel", dump=DUMP_N)
    plsc.subcore_barrier()
    _flush(acc, s2_h, base, HALF, sid)


_counts_call = pl.kernel(
    _counts_body,
    out_type=jax.ShapeDtypeStruct((NN, L), BF),
    mesh=_MESH,
    scratch_types=_scratch_common(ACC_N_ROWS, BF) + [
        pltpu.VMEM_SHARED((R, L), BF)],
    compiler_params=_SC_PARAMS,
)

_l1_call = pl.kernel(
    _l1_body,
    out_type=jax.ShapeDtypeStruct((NN1 * R, L), BF),
    mesh=_MESH,
    scratch_types=_scratch_common(ACC1_ROWS, BF) + [
        pltpu.VMEM_SHARED((NN, L), BF)],
    compiler_params=_SC_PARAMS,
)

_l2_call = pl.kernel(
    _l2_body,
    out_type=jax.ShapeDtypeStruct((NN, L), jnp.float32),
    mesh=_MESH,
    scratch_types=_scratch_common(ACC_N_ROWS, jnp.float32),
    compiler_params=_SC_PARAMS,
)


BROW = 2000


def _dense1_body(x_ref, s1_ref, cnt_ref, w1_ref, r1_ref, b1_ref, w2c_ref,
                 h_ref, zm_ref):
    xb = x_ref[...]
    s1 = s1_ref[...].astype(jnp.float32)
    cb = jnp.maximum(cnt_ref[...][:, :R].astype(jnp.float32), 1.0)
    acc = jnp.dot(xb, r1_ref[...], preferred_element_type=jnp.float32)
    acc = acc + b1_ref[...]
    for r in range(R):
        pr = jnp.dot(s1[:, r * INCH:(r + 1) * INCH],
                     w1_ref[...][r * INCH:(r + 1) * INCH, :],
                     preferred_element_type=jnp.float32)
        acc = acc + pr / cb[:, r:r + 1]
    h = jnp.maximum(acc, 0.0)
    h_ref[...] = h
    z = jnp.dot(h, w2c_ref[...], preferred_element_type=jnp.float32)
    zt = jnp.concatenate([z] * R, axis=1)
    lane = lax.broadcasted_iota(jnp.int32, (BROW, R * L), 1)
    keep = (lane % L) // OUTCH == lane // L
    zm_ref[...] = jnp.where(keep, zt, 0.0)


def _dense2_body(h_ref, s2_ref, cnt_ref, r2_ref, b2_ref, out_ref):
    h = h_ref[...]
    s2 = s2_ref[...].astype(jnp.float32)
    cb = jnp.maximum(cnt_ref[...][:, :R].astype(jnp.float32), 1.0)
    acc = jnp.dot(h, r2_ref[...], preferred_element_type=jnp.float32)
    acc = acc + b2_ref[...]
    for r in range(R):
        acc = acc + s2[:, OUTCH * r:OUTCH * (r + 1)] / cb[:, r:r + 1]
    m = jnp.max(acc, axis=1, keepdims=True)
    ex = jnp.exp(acc - m)
    out_ref[...] = acc - m - jnp.log(jnp.sum(ex, axis=1, keepdims=True))


def _full_spec(shape):
    return pl.BlockSpec(shape, lambda i: (0, 0))


_dense1_call = pl.pallas_call(
    _dense1_body,
    grid=(NN // BROW,),
    in_specs=[
        pl.BlockSpec((BROW, INCH), lambda i: (i, 0)),
        pl.BlockSpec((BROW, R * INCH), lambda i: (i, 0)),
        pl.BlockSpec((BROW, L), lambda i: (i, 0)),
        _full_spec((R * INCH, HID)),
        _full_spec((INCH, HID)),
        _full_spec((1, HID)),
        _full_spec((HID, R * OUTCH)),
    ],
    out_specs=[
        pl.BlockSpec((BROW, HID), lambda i: (i, 0)),
        pl.BlockSpec((BROW, R * L), lambda i: (i, 0)),
    ],
    out_shape=[
        jax.ShapeDtypeStruct((NN, HID), jnp.float32),
        jax.ShapeDtypeStruct((NN, R * L), jnp.float32),
    ],
)

_dense2_call = pl.pallas_call(
    _dense2_body,
    grid=(NN // BROW,),
    in_specs=[
        pl.BlockSpec((BROW, HID), lambda i: (i, 0)),
        pl.BlockSpec((BROW, L), lambda i: (i, 0)),
        pl.BlockSpec((BROW, L), lambda i: (i, 0)),
        _full_spec((HID, OUTCH)),
        _full_spec((1, OUTCH)),
    ],
    out_specs=pl.BlockSpec((BROW, OUTCH), lambda i: (i, 0)),
    out_shape=jax.ShapeDtypeStruct((NN, OUTCH), jnp.float32),
)


def kernel(x, edge_index, edge_type, W1, root1, b1, W2, root2, b2):
    pad = NE_P - NE
    src_p = jnp.concatenate([edge_index[0], jnp.zeros((pad,), jnp.int32)])
    dst_p = jnp.concatenate([edge_index[1], jnp.full((pad,), NN, jnp.int32)])
    et_p = jnp.concatenate([edge_type, jnp.zeros((pad,), jnp.int32)])
    e3 = jnp.stack([src_p.reshape(NBLKS_P, BLK), dst_p.reshape(NBLKS_P, BLK),
                    et_p.reshape(NBLKS_P, BLK)], axis=1).reshape(NBLKS_P, EROW)

    onehot = jnp.eye(R, L, dtype=BF)
    zeros128b = jnp.zeros((128, L), BF)
    zeros128f = jnp.zeros((128, L), jnp.float32)

    cnt = _counts_call(e3, onehot, zeros128b)
    s1 = _l1_call(e3, x.astype(BF), zeros128b)

    w1cat = W1.reshape(R * INCH, HID)
    w2cat = jnp.transpose(W2, (1, 0, 2)).reshape(HID, R * OUTCH)
    h, zm = _dense1_call(x, s1.reshape(NN1, R * INCH)[:NN], cnt, w1cat, root1,
                         b1.reshape(1, HID), w2cat)
    s2 = _l2_call(e3, zm.reshape(NN * R, L), zeros128f)
    out = _dense2_call(h, s2, cnt, root2, b2.reshape(1, OUTCH))
    return out


# speedup vs baseline: 17.6528x; 1.0857x over previous
import jax
import jax.numpy as jnp
from jax import lax
from jax.experimental import pallas as pl
from jax.experimental.pallas import tpu as pltpu
from jax.experimental.pallas import tpu_sc as plsc

NN = 100000
NE = 3200000
R = 8
INCH = 16
HID = 32
OUTCH = 2

NC = 2
NS = 16
L = 16

BLK = 128
NBUF = 6
NBLKS_P = 25152
NE_P = NBLKS_P * BLK
PER_SUB = NBLKS_P // NS
EROW = 3 * BLK

BF = jnp.bfloat16

HALF = NN // NC
ACC_N_ROWS = 65536
DUMP_N = HALF

CHUNK1 = 16672
NCHUNK1 = 6
NN1 = CHUNK1 * NCHUNK1
ACC1_ROWS = 135168
DUMP1 = CHUNK1 * R

_MESH = plsc.VectorSubcoreMesh(core_axis_name="c", subcore_axis_name="s")
_SC_PARAMS = pltpu.CompilerParams(use_tc_tiling_on_sc=False)


def _edge_scan(sid, base, hi, *, e3_h, table, acc, bufs, sems,
               smode, gmode, dump):
    ebuf, gidx, sidx, rows = bufs
    start = sid * PER_SUB

    def group_body(g, carry):
        b0 = start + g * NBUF
        lds, gds, sds = [], [], []
        for u in range(NBUF):
            d = pltpu.make_async_copy(e3_h.at[b0 + u], ebuf[u],
                                      sems.at[u, 0])
            d.start()
            lds.append(d)
        for u in range(NBUF):
            lds[u].wait()
            for j in range(BLK // L):
                sl = pl.ds(j * L, L)
                dv = ebuf[u][pl.ds(BLK + j * L, L)]
                ok = (dv >= base) & (dv < base + hi)
                if smode == "node_rel":
                    ev = ebuf[u][pl.ds(2 * BLK + j * L, L)]
                    s = jnp.where(ok, (dv - base) * R + ev, dump)
                else:
                    s = jnp.where(ok, dv - base, dump)
                sidx[u][sl] = s
                if gmode == "et":
                    gidx[u][sl] = ebuf[u][pl.ds(2 * BLK + j * L, L)]
                elif gmode == "src":
                    gidx[u][sl] = ebuf[u][pl.ds(j * L, L)]
                else:
                    gidx[u][sl] = (ebuf[u][pl.ds(j * L, L)] * R
                                   + ebuf[u][pl.ds(2 * BLK + j * L, L)])
            d = pltpu.make_async_copy(table.at[gidx[u]], rows[u],
                                      sems.at[u, 1])
            d.start()
            gds.append(d)
        for u in range(NBUF):
            gds[u].wait()
            d = pltpu.make_async_copy(rows[u], acc.at[sidx[u]],
                                      sems.at[u, 2])
            d.start(add=True)
            sds.append(d)
        for u in range(NBUF):
            sds[u].wait()
        return carry

    lax.fori_loop(0, PER_SUB // NBUF, group_body, 0)


def _zero_acc(zb, acc, sid, per_tile):
    def zchunk(k, carry):
        pltpu.sync_copy(zb, acc.at[pl.ds(sid * per_tile + k * 128, 128)])
        return carry

    lax.fori_loop(0, per_tile // 128, zchunk, 0)


_ZROWS_N = 4096
_ZROWS_1 = 8448


def _striped(src_fn, dst_fn, total, sid):
    s_main = (total // NS) & ~7
    last = total - s_main * (NS - 1)

    @pl.when(sid < NS - 1)
    def _():
        pltpu.sync_copy(src_fn(sid * s_main, s_main),
                        dst_fn(sid * s_main, s_main))

    @pl.when(sid == NS - 1)
    def _():
        pltpu.sync_copy(src_fn((NS - 1) * s_main, last),
                        dst_fn((NS - 1) * s_main, last))


def _flush(acc, out_h, out_base, total, sid):
    _striped(lambda o, n: acc.at[pl.ds(o, n)],
             lambda o, n: out_h.at[pl.ds(out_base + o, n)], total, sid)


def _stage(in_h, spm, total, sid):
    _striped(lambda o, n: in_h.at[pl.ds(o, n)],
             lambda o, n: spm.at[pl.ds(o, n)], total, sid)


def _scratch_common(acc_rows, dt):
    return [
        pltpu.VMEM_SHARED((acc_rows, L), dt),
        pltpu.VMEM((128, L), dt),
        [pltpu.VMEM((EROW,), jnp.int32) for _ in range(NBUF)],
        [pltpu.VMEM((BLK,), jnp.int32) for _ in range(NBUF)],
        [pltpu.VMEM((BLK,), jnp.int32) for _ in range(NBUF)],
        [pltpu.VMEM((BLK, L), dt) for _ in range(NBUF)],
        pltpu.SemaphoreType.DMA((NBUF, 3)),
    ]


def _counts_body(e3_h, onehot_h, zeros_h, cnt_h,
                 acc, zb, ebuf, gidx, sidx, rows, sems, pm):
    cid = lax.axis_index("c")
    sid = lax.axis_index("s")
    base = cid * HALF
    pltpu.sync_copy(zeros_h, zb)

    @pl.when(sid == 0)
    def _():
        pltpu.sync_copy(onehot_h, pm)

    _zero_acc(zb, acc, sid, _ZROWS_N)
    plsc.subcore_barrier()
    _edge_scan(sid, base, HALF, e3_h=e3_h, table=pm, acc=acc,
               bufs=(ebuf, gidx, sidx, rows), sems=sems,
               smode="node", gmode="et", dump=DUMP_N)
    plsc.subcore_barrier()
    _flush(acc, cnt_h, base, HALF, sid)


def _l1_body(e3_h, x_h, zeros_h, s1_h,
             acc, zb, ebuf, gidx, sidx, rows, sems, xs):
    cid = lax.axis_index("c")
    sid = lax.axis_index("s")
    pltpu.sync_copy(zeros_h, zb)
    _stage(x_h, xs, NN, sid)

    def pass_body(p, carry):
        chunk = cid * (NCHUNK1 // NC) + p
        base = chunk * CHUNK1
        _zero_acc(zb, acc, sid, _ZROWS_1)
        plsc.subcore_barrier()
        _edge_scan(sid, base, CHUNK1, e3_h=e3_h, table=xs, acc=acc,
                   bufs=(ebuf, gidx, sidx, rows), sems=sems,
                   smode="node_rel", gmode="src", dump=DUMP1)
        plsc.subcore_barrier()
        _flush(acc, s1_h, chunk * CHUNK1 * R, CHUNK1 * R, sid)
        plsc.subcore_barrier()
        return carry

    lax.fori_loop(0, NCHUNK1 // NC, pass_body, 0)


def _l2_body(e3_h, zm_h, zeros_h, s2_h,
             acc, zb, ebuf, gidx, sidx, rows, sems):
    cid = lax.axis_index("c")
    sid = lax.axis_index("s")
    base = cid * HALF
    pltpu.sync_copy(zeros_h, zb)
    _zero_acc(zb, acc, sid, _ZROWS_N)
    plsc.subcore_barrier()
    _edge_scan(sid, base, HALF, e3_h=e3_h, table=zm_h, acc=acc,
               bufs=(ebuf, gidx, sidx, rows), sems=sems,
               smode="node", gmode="src_rel", dump=DUMP_N)
    plsc.subcore_barrier()
    _flush(acc, s2_h, base, HALF, sid)


_counts_call = pl.kernel(
    _counts_body,
    out_type=jax.ShapeDtypeStruct((NN, L), BF),
    mesh=_MESH,
    scratch_types=_scratch_common(ACC_N_ROWS, BF) + [
        pltpu.VMEM_SHARED((R, L), BF)],
    compiler_params=_SC_PARAMS,
)

_l1_call = pl.kernel(
    _l1_body,
    out_type=jax.ShapeDtypeStruct((NN1 * R, L), BF),
    mesh=_MESH,
    scratch_types=_scratch_common(ACC1_ROWS, BF) + [
        pltpu.VMEM_SHARED((NN, L), BF)],
    compiler_params=_SC_PARAMS,
)

_l2_call = pl.kernel(
    _l2_body,
    out_type=jax.ShapeDtypeStruct((NN, L), BF),
    mesh=_MESH,
    scratch_types=_scratch_common(ACC_N_ROWS, BF),
    compiler_params=_SC_PARAMS,
)


BROW = 2000


def _dense1_body(x_ref, s1_ref, cnt_ref, w1_ref, r1_ref, b1_ref, w2c_ref,
                 h_ref, zm_ref):
    xb = x_ref[...]
    s1 = s1_ref[...].astype(jnp.float32)
    cb = jnp.maximum(cnt_ref[...][:, :R].astype(jnp.float32), 1.0)
    acc = jnp.dot(xb, r1_ref[...], preferred_element_type=jnp.float32)
    acc = acc + b1_ref[...]
    for r in range(R):
        pr = jnp.dot(s1[:, r * INCH:(r + 1) * INCH],
                     w1_ref[...][r * INCH:(r + 1) * INCH, :],
                     preferred_element_type=jnp.float32)
        acc = acc + pr / cb[:, r:r + 1]
    h = jnp.maximum(acc, 0.0)
    h_ref[...] = h
    z = jnp.dot(h, w2c_ref[...], preferred_element_type=jnp.float32)
    zt = jnp.concatenate([z] * R, axis=1)
    lane = lax.broadcasted_iota(jnp.int32, (BROW, R * L), 1)
    keep = (lane % L) // OUTCH == lane // L
    zm_ref[...] = jnp.where(keep, zt, 0.0).astype(BF)


def _dense2_body(h_ref, s2_ref, cnt_ref, r2_ref, b2_ref, out_ref):
    h = h_ref[...]
    s2 = s2_ref[...].astype(jnp.float32)
    cb = jnp.maximum(cnt_ref[...][:, :R].astype(jnp.float32), 1.0)
    acc = jnp.dot(h, r2_ref[...], preferred_element_type=jnp.float32)
    acc = acc + b2_ref[...]
    for r in range(R):
        acc = acc + s2[:, OUTCH * r:OUTCH * (r + 1)] / cb[:, r:r + 1]
    m = jnp.max(acc, axis=1, keepdims=True)
    ex = jnp.exp(acc - m)
    out_ref[...] = acc - m - jnp.log(jnp.sum(ex, axis=1, keepdims=True))


def _full_spec(shape):
    return pl.BlockSpec(shape, lambda i: (0, 0))


_dense1_call = pl.pallas_call(
    _dense1_body,
    grid=(NN // BROW,),
    in_specs=[
        pl.BlockSpec((BROW, INCH), lambda i: (i, 0)),
        pl.BlockSpec((BROW, R * INCH), lambda i: (i, 0)),
        pl.BlockSpec((BROW, L), lambda i: (i, 0)),
        _full_spec((R * INCH, HID)),
        _full_spec((INCH, HID)),
        _full_spec((1, HID)),
        _full_spec((HID, R * OUTCH)),
    ],
    out_specs=[
        pl.BlockSpec((BROW, HID), lambda i: (i, 0)),
        pl.BlockSpec((BROW, R * L), lambda i: (i, 0)),
    ],
    out_shape=[
        jax.ShapeDtypeStruct((NN, HID), jnp.float32),
        jax.ShapeDtypeStruct((NN, R * L), BF),
    ],
)

_dense2_call = pl.pallas_call(
    _dense2_body,
    grid=(NN // BROW,),
    in_specs=[
        pl.BlockSpec((BROW, HID), lambda i: (i, 0)),
        pl.BlockSpec((BROW, L), lambda i: (i, 0)),
        pl.BlockSpec((BROW, L), lambda i: (i, 0)),
        _full_spec((HID, OUTCH)),
        _full_spec((1, OUTCH)),
    ],
    out_specs=pl.BlockSpec((BROW, OUTCH), lambda i: (i, 0)),
    out_shape=jax.ShapeDtypeStruct((NN, OUTCH), jnp.float32),
)


def kernel(x, edge_index, edge_type, W1, root1, b1, W2, root2, b2):
    pad = NE_P - NE
    src_p = jnp.concatenate([edge_index[0], jnp.zeros((pad,), jnp.int32)])
    dst_p = jnp.concatenate([edge_index[1], jnp.full((pad,), NN, jnp.int32)])
    et_p = jnp.concatenate([edge_type, jnp.zeros((pad,), jnp.int32)])
    e3 = jnp.stack([src_p.reshape(NBLKS_P, BLK), dst_p.reshape(NBLKS_P, BLK),
                    et_p.reshape(NBLKS_P, BLK)], axis=1).reshape(NBLKS_P, EROW)

    onehot = jnp.eye(R, L, dtype=BF)
    zeros128b = jnp.zeros((128, L), BF)
    zeros128f = jnp.zeros((128, L), jnp.float32)

    cnt = _counts_call(e3, onehot, zeros128b)
    s1 = _l1_call(e3, x.astype(BF), zeros128b)

    w1cat = W1.reshape(R * INCH, HID)
    w2cat = jnp.transpose(W2, (1, 0, 2)).reshape(HID, R * OUTCH)
    h, zm = _dense1_call(x, s1.reshape(NN1, R * INCH)[:NN], cnt, w1cat, root1,
                         b1.reshape(1, HID), w2cat)
    s2 = _l2_call(e3, zm.reshape(NN * R, L), zeros128b)
    out = _dense2_call(h, s2, cnt, root2, b2.reshape(1, OUTCH))
    return out


# speedup vs baseline: 23.0951x; 1.3083x over previous
import jax
import jax.numpy as jnp
from jax import lax
from jax.experimental import pallas as pl
from jax.experimental.pallas import tpu as pltpu
from jax.experimental.pallas import tpu_sc as plsc

NN = 100000
NE = 3200000
R = 8
INCH = 16
HID = 32
OUTCH = 2

NC = 2
NS = 16
L = 16

BLK = 128
NBUF = 6
NBLKS_P = 25152
NE_P = NBLKS_P * BLK
PER_SUB = NBLKS_P // NS
EROW = 3 * BLK

BF = jnp.bfloat16

HALF = NN // NC
ACC_N_ROWS = 65536
DUMP_N = HALF

CHUNK1 = 25008
NCHUNK1 = 4
NN1 = CHUNK1 * NCHUNK1
ACC1_ROWS = 200704
DUMP1 = CHUNK1 * R

_MESH = plsc.VectorSubcoreMesh(core_axis_name="c", subcore_axis_name="s")
_SC_PARAMS = pltpu.CompilerParams(use_tc_tiling_on_sc=False)


def _edge_scan(sid, base, hi, *, e3_h, table, acc, bufs, sems,
               smode, gmode, dump):
    ebuf, gidx, sidx, rows = bufs
    start = sid * PER_SUB

    def group_body(g, carry):
        b0 = start + g * NBUF
        lds, gds, sds = [], [], []
        for u in range(NBUF):
            d = pltpu.make_async_copy(e3_h.at[b0 + u], ebuf[u],
                                      sems.at[u, 0])
            d.start()
            lds.append(d)
        for u in range(NBUF):
            lds[u].wait()
            for j in range(BLK // L):
                sl = pl.ds(j * L, L)
                dv = ebuf[u][pl.ds(BLK + j * L, L)]
                ok = (dv >= base) & (dv < base + hi)
                if smode == "node_rel":
                    ev = ebuf[u][pl.ds(2 * BLK + j * L, L)]
                    s = jnp.where(ok, (dv - base) * R + ev, dump)
                else:
                    s = jnp.where(ok, dv - base, dump)
                sidx[u][sl] = s
                if gmode == "et":
                    gidx[u][sl] = ebuf[u][pl.ds(2 * BLK + j * L, L)]
                elif gmode == "src":
                    gidx[u][sl] = ebuf[u][pl.ds(j * L, L)]
                else:
                    gidx[u][sl] = (ebuf[u][pl.ds(j * L, L)] * R
                                   + ebuf[u][pl.ds(2 * BLK + j * L, L)])
            d = pltpu.make_async_copy(table.at[gidx[u]], rows[u],
                                      sems.at[u, 1])
            d.start()
            gds.append(d)
        for u in range(NBUF):
            gds[u].wait()
            d = pltpu.make_async_copy(rows[u], acc.at[sidx[u]],
                                      sems.at[u, 2])
            d.start(add=True)
            sds.append(d)
        for u in range(NBUF):
            sds[u].wait()
        return carry

    lax.fori_loop(0, PER_SUB // NBUF, group_body, 0)


def _zero_acc(zb, acc, sid, per_tile):
    def zchunk(k, carry):
        pltpu.sync_copy(zb, acc.at[pl.ds(sid * per_tile + k * 128, 128)])
        return carry

    lax.fori_loop(0, per_tile // 128, zchunk, 0)


_ZROWS_N = 4096
_ZROWS_1 = 12544


def _striped(src_fn, dst_fn, total, sid):
    s_main = (total // NS) & ~7
    last = total - s_main * (NS - 1)

    @pl.when(sid < NS - 1)
    def _():
        pltpu.sync_copy(src_fn(sid * s_main, s_main),
                        dst_fn(sid * s_main, s_main))

    @pl.when(sid == NS - 1)
    def _():
        pltpu.sync_copy(src_fn((NS - 1) * s_main, last),
                        dst_fn((NS - 1) * s_main, last))


def _flush(acc, out_h, out_base, total, sid):
    _striped(lambda o, n: acc.at[pl.ds(o, n)],
             lambda o, n: out_h.at[pl.ds(out_base + o, n)], total, sid)


def _stage(in_h, spm, total, sid):
    _striped(lambda o, n: in_h.at[pl.ds(o, n)],
             lambda o, n: spm.at[pl.ds(o, n)], total, sid)


def _scratch_common(acc_rows, dt):
    return [
        pltpu.VMEM_SHARED((acc_rows, L), dt),
        pltpu.VMEM((128, L), dt),
        [pltpu.VMEM((EROW,), jnp.int32) for _ in range(NBUF)],
        [pltpu.VMEM((BLK,), jnp.int32) for _ in range(NBUF)],
        [pltpu.VMEM((BLK,), jnp.int32) for _ in range(NBUF)],
        [pltpu.VMEM((BLK, L), dt) for _ in range(NBUF)],
        pltpu.SemaphoreType.DMA((NBUF, 3)),
    ]


def _counts_body(e3_h, onehot_h, zeros_h, cnt_h,
                 acc, zb, ebuf, gidx, sidx, rows, sems, pm):
    cid = lax.axis_index("c")
    sid = lax.axis_index("s")
    base = cid * HALF
    pltpu.sync_copy(zeros_h, zb)

    @pl.when(sid == 0)
    def _():
        pltpu.sync_copy(onehot_h, pm)

    _zero_acc(zb, acc, sid, _ZROWS_N)
    plsc.subcore_barrier()
    _edge_scan(sid, base, HALF, e3_h=e3_h, table=pm, acc=acc,
               bufs=(ebuf, gidx, sidx, rows), sems=sems,
               smode="node", gmode="et", dump=DUMP_N)
    plsc.subcore_barrier()
    _flush(acc, cnt_h, base, HALF, sid)


def _l1_body(e3_h, x_h, zeros_h, s1_h,
             acc, zb, ebuf, gidx, sidx, rows, sems):
    cid = lax.axis_index("c")
    sid = lax.axis_index("s")
    pltpu.sync_copy(zeros_h, zb)

    def pass_body(p, carry):
        chunk = cid * (NCHUNK1 // NC) + p
        base = chunk * CHUNK1
        _zero_acc(zb, acc, sid, _ZROWS_1)
        plsc.subcore_barrier()
        _edge_scan(sid, base, CHUNK1, e3_h=e3_h, table=x_h, acc=acc,
                   bufs=(ebuf, gidx, sidx, rows), sems=sems,
                   smode="node_rel", gmode="src", dump=DUMP1)
        plsc.subcore_barrier()
        _flush(acc, s1_h, chunk * CHUNK1 * R, CHUNK1 * R, sid)
        plsc.subcore_barrier()
        return carry

    lax.fori_loop(0, NCHUNK1 // NC, pass_body, 0)


def _l2_body(e3_h, zm_h, zeros_h, s2_h,
             acc, zb, ebuf, gidx, sidx, rows, sems):
    cid = lax.axis_index("c")
    sid = lax.axis_index("s")
    base = cid * HALF
    pltpu.sync_copy(zeros_h, zb)
    _zero_acc(zb, acc, sid, _ZROWS_N)
    plsc.subcore_barrier()
    _edge_scan(sid, base, HALF, e3_h=e3_h, table=zm_h, acc=acc,
               bufs=(ebuf, gidx, sidx, rows), sems=sems,
               smode="node", gmode="src_rel", dump=DUMP_N)
    plsc.subcore_barrier()
    _flush(acc, s2_h, base, HALF, sid)


_counts_call = pl.kernel(
    _counts_body,
    out_type=jax.ShapeDtypeStruct((NN, L), BF),
    mesh=_MESH,
    scratch_types=_scratch_common(ACC_N_ROWS, BF) + [
        pltpu.VMEM_SHARED((R, L), BF)],
    compiler_params=_SC_PARAMS,
)

_l1_call = pl.kernel(
    _l1_body,
    out_type=jax.ShapeDtypeStruct((NN1 * R, L), BF),
    mesh=_MESH,
    scratch_types=_scratch_common(ACC1_ROWS, BF),
    compiler_params=_SC_PARAMS,
)

_l2_call = pl.kernel(
    _l2_body,
    out_type=jax.ShapeDtypeStruct((NN, L), BF),
    mesh=_MESH,
    scratch_types=_scratch_common(ACC_N_ROWS, BF),
    compiler_params=_SC_PARAMS,
)


BROW = 2000


def _dense1_body(x_ref, s1_ref, cnt_ref, w1_ref, r1_ref, b1_ref, w2c_ref,
                 h_ref, zm_ref):
    xb = x_ref[...]
    s1 = s1_ref[...].astype(jnp.float32)
    cb = jnp.maximum(cnt_ref[...][:, :R].astype(jnp.float32), 1.0)
    acc = jnp.dot(xb, r1_ref[...], preferred_element_type=jnp.float32)
    acc = acc + b1_ref[...]
    for r in range(R):
        pr = jnp.dot(s1[:, r * INCH:(r + 1) * INCH],
                     w1_ref[...][r * INCH:(r + 1) * INCH, :],
                     preferred_element_type=jnp.float32)
        acc = acc + pr / cb[:, r:r + 1]
    h = jnp.maximum(acc, 0.0)
    h_ref[...] = h
    z = jnp.dot(h, w2c_ref[...], preferred_element_type=jnp.float32)
    zt = jnp.concatenate([z] * R, axis=1)
    lane = lax.broadcasted_iota(jnp.int32, (BROW, R * L), 1)
    keep = (lane % L) // OUTCH == lane // L
    zm_ref[...] = jnp.where(keep, zt, 0.0).astype(BF)


def _dense2_body(h_ref, s2_ref, cnt_ref, r2_ref, b2_ref, out_ref):
    h = h_ref[...]
    s2 = s2_ref[...].astype(jnp.float32)
    cb = jnp.maximum(cnt_ref[...][:, :R].astype(jnp.float32), 1.0)
    acc = jnp.dot(h, r2_ref[...], preferred_element_type=jnp.float32)
    acc = acc + b2_ref[...]
    for r in range(R):
        acc = acc + s2[:, OUTCH * r:OUTCH * (r + 1)] / cb[:, r:r + 1]
    m = jnp.max(acc, axis=1, keepdims=True)
    ex = jnp.exp(acc - m)
    out_ref[...] = acc - m - jnp.log(jnp.sum(ex, axis=1, keepdims=True))


def _full_spec(shape):
    return pl.BlockSpec(shape, lambda i: (0, 0))


_dense1_call = pl.pallas_call(
    _dense1_body,
    grid=(NN // BROW,),
    in_specs=[
        pl.BlockSpec((BROW, INCH), lambda i: (i, 0)),
        pl.BlockSpec((BROW, R * INCH), lambda i: (i, 0)),
        pl.BlockSpec((BROW, L), lambda i: (i, 0)),
        _full_spec((R * INCH, HID)),
        _full_spec((INCH, HID)),
        _full_spec((1, HID)),
        _full_spec((HID, R * OUTCH)),
    ],
    out_specs=[
        pl.BlockSpec((BROW, HID), lambda i: (i, 0)),
        pl.BlockSpec((BROW, R * L), lambda i: (i, 0)),
    ],
    out_shape=[
        jax.ShapeDtypeStruct((NN, HID), jnp.float32),
        jax.ShapeDtypeStruct((NN, R * L), BF),
    ],
)

_dense2_call = pl.pallas_call(
    _dense2_body,
    grid=(NN // BROW,),
    in_specs=[
        pl.BlockSpec((BROW, HID), lambda i: (i, 0)),
        pl.BlockSpec((BROW, L), lambda i: (i, 0)),
        pl.BlockSpec((BROW, L), lambda i: (i, 0)),
        _full_spec((HID, OUTCH)),
        _full_spec((1, OUTCH)),
    ],
    out_specs=pl.BlockSpec((BROW, OUTCH), lambda i: (i, 0)),
    out_shape=jax.ShapeDtypeStruct((NN, OUTCH), jnp.float32),
)


def kernel(x, edge_index, edge_type, W1, root1, b1, W2, root2, b2):
    pad = NE_P - NE
    src_p = jnp.concatenate([edge_index[0], jnp.zeros((pad,), jnp.int32)])
    dst_p = jnp.concatenate([edge_index[1], jnp.full((pad,), NN, jnp.int32)])
    et_p = jnp.concatenate([edge_type, jnp.zeros((pad,), jnp.int32)])
    e3 = jnp.stack([src_p.reshape(NBLKS_P, BLK), dst_p.reshape(NBLKS_P, BLK),
                    et_p.reshape(NBLKS_P, BLK)], axis=1).reshape(NBLKS_P, EROW)

    onehot = jnp.eye(R, L, dtype=BF)
    zeros128b = jnp.zeros((128, L), BF)
    zeros128f = jnp.zeros((128, L), jnp.float32)

    cnt = _counts_call(e3, onehot, zeros128b)
    s1 = _l1_call(e3, x.astype(BF), zeros128b)

    w1cat = W1.reshape(R * INCH, HID)
    w2cat = jnp.transpose(W2, (1, 0, 2)).reshape(HID, R * OUTCH)
    h, zm = _dense1_call(x, s1.reshape(NN1, R * INCH)[:NN], cnt, w1cat, root1,
                         b1.reshape(1, HID), w2cat)
    s2 = _l2_call(e3, zm.reshape(NN * R, L), zeros128b)
    out = _dense2_call(h, s2, cnt, root2, b2.reshape(1, OUTCH))
    return out
